# baseline (speedup 1.0000x reference)
import functools

import jax
import jax.numpy as jnp
from jax import lax
from jax.experimental import pallas as pl
from jax.experimental.pallas import tpu as pltpu
from jax.experimental.pallas import tpu_sc as plsc


def _embedding_gather(ids_t, table, *, num_workers):
    n_rows, n_tok = ids_t.shape
    dim = table.shape[1]
    chunk = n_tok // num_workers
    mesh = plsc.VectorSubcoreMesh(core_axis_name="c", subcore_axis_name="s")

    @functools.partial(
        pl.kernel,
        out_type=jax.ShapeDtypeStruct((n_rows, dim, n_tok), jnp.float32),
        mesh=mesh,
        scratch_types=[
            pltpu.VMEM((n_rows, chunk), jnp.int32),
            [pltpu.VMEM((chunk, dim), jnp.float32) for _ in range(2)],
            [pltpu.VMEM((dim, chunk), jnp.float32) for _ in range(2)],
            [pltpu.SemaphoreType.DMA for _ in range(2)],
            [pltpu.SemaphoreType.DMA for _ in range(2)],
        ],
        compiler_params=pltpu.CompilerParams(
            use_tc_tiling_on_sc=False, needs_layout_passes=False
        ),
    )
    def k(ids_hbm, table_hbm, out_hbm, idx_v, rows, trans, g_sems, w_sems):
        wid = lax.axis_index("s") * 2 + lax.axis_index("c")
        base = wid * chunk
        pltpu.sync_copy(ids_hbm.at[:, pl.ds(base, chunk)], idx_v)

        def transpose_chunk(src, dst):
            def tbody(g, _):
                tok = lax.iota(jnp.int32, 16) + g * 16
                for c in range(dim):
                    v = plsc.load_gather(
                        src, [tok, jnp.full((16,), c, jnp.int32)]
                    )
                    dst[c, pl.ds(g * 16, 16)] = v
                return 0

            lax.fori_loop(0, chunk // 16, tbody, 0)

        def body(g, _):
            j0 = g * 2
            j1 = j0 + 1
            gd0 = pltpu.async_copy(
                table_hbm.at[idx_v.at[j0]], rows[0], g_sems[0]
            )
            gd1 = pltpu.async_copy(
                table_hbm.at[idx_v.at[j1]], rows[1], g_sems[1]
            )
            gd0.wait()
            transpose_chunk(rows[0], trans[0])
            wd0 = pltpu.async_copy(
                trans[0], out_hbm.at[j0, :, pl.ds(base, chunk)], w_sems[0]
            )
            gd1.wait()
            transpose_chunk(rows[1], trans[1])
            wd1 = pltpu.async_copy(
                trans[1], out_hbm.at[j1, :, pl.ds(base, chunk)], w_sems[1]
            )
            wd0.wait()
            wd1.wait()
            return 0

        lax.fori_loop(0, n_rows // 2, body, 0)

    return k(ids_t, table)


def kernel(token_ids, table):
    ids_t = token_ids.T.astype(jnp.int32)
    out_t = _embedding_gather(ids_t, table, num_workers=32)
    return out_t.transpose(2, 0, 1)
